# Initial kernel scaffold; baseline (speedup 1.0000x reference)
import jax
import jax.numpy as jnp
from jax.experimental import pallas as pl


def kernel(x, edge_index, Ws, bs, gammas, betas):
    raise NotImplementedError("write your pallas kernel here")


# speedup vs baseline: 8.3734x; 8.3734x over previous
import functools

import jax
import jax.numpy as jnp
from jax import lax
from jax.experimental import pallas as pl
from jax.experimental.pallas import tpu as pltpu
from jax.experimental.pallas import tpu_sc as plsc

N = 10000
E = 320000
D = 64
NUM_CLASSES = 47
EPS_BN = 1e-5

NC, NS = 2, 16
NW = NC * NS
EPW = E // NW
CH = 80
NCHUNK = EPW // CH
RPT = 624
TAIL = N - NS * RPT


def _sc_aggregate_call(y, src_r, dst_r, zeros):
  mesh = plsc.VectorSubcoreMesh(core_axis_name="c", subcore_axis_name="s")

  @functools.partial(
      pl.kernel,
      out_type=jax.ShapeDtypeStruct((NC, N, D), jnp.float32),
      mesh=mesh,
      scratch_types=[
          pltpu.VMEM((NCHUNK, CH), jnp.int32),
          pltpu.VMEM((NCHUNK, CH), jnp.int32),
          pltpu.VMEM((CH, D), jnp.float32),
          pltpu.VMEM_SHARED((N, D), jnp.float32),
      ],
      compiler_params=pltpu.CompilerParams(use_tc_tiling_on_sc=False),
  )
  def agg(y_hbm, src_hbm, dst_hbm, zeros_hbm, out_hbm, src_v, dst_v, rows_v,
          acc_s):
    cid = lax.axis_index("c")
    sid = lax.axis_index("s")
    wid = sid * NC + cid

    pltpu.sync_copy(zeros_hbm.at[pl.ds(sid * RPT, RPT)],
                    acc_s.at[pl.ds(sid * RPT, RPT)])

    @pl.when(sid == NS - 1)
    def _seed_tail():
      pltpu.sync_copy(zeros_hbm.at[pl.ds(NS * RPT, TAIL)],
                      acc_s.at[pl.ds(NS * RPT, TAIL)])

    pltpu.sync_copy(src_hbm.at[wid], src_v)
    pltpu.sync_copy(dst_hbm.at[wid], dst_v)
    plsc.subcore_barrier()

    def step(j, carry):
      pltpu.sync_copy(y_hbm.at[src_v.at[j]], rows_v)
      pltpu.sync_copy(rows_v, acc_s.at[dst_v.at[j]], add=True)
      return carry

    lax.fori_loop(0, NCHUNK, step, 0, unroll=False)

    plsc.subcore_barrier()
    pltpu.sync_copy(acc_s.at[pl.ds(sid * RPT, RPT)],
                    out_hbm.at[cid, pl.ds(sid * RPT, RPT)])

    @pl.when(sid == NS - 1)
    def _write_tail():
      pltpu.sync_copy(acc_s.at[pl.ds(NS * RPT, TAIL)],
                      out_hbm.at[cid, pl.ds(NS * RPT, TAIL)])

  return agg(y, src_r, dst_r, zeros)


def _mm_body(x_ref, w_ref, o_ref):
  o_ref[...] = lax.dot_general(
      x_ref[...], w_ref[...], (((1,), (1,)), ((), ())),
      preferred_element_type=jnp.float32, precision=lax.Precision.HIGHEST)


def _input_matmul(x, w):
  return pl.pallas_call(
      _mm_body,
      out_shape=jax.ShapeDtypeStruct((N, D), jnp.float32),
  )(x, w)


def _layer_body(y_ref, acc_ref, b_ref, g_ref, be_ref, w_ref, o_ref):
  z = y_ref[...] + acc_ref[0] + acc_ref[1] + b_ref[...]
  mean = jnp.mean(z, axis=0, keepdims=True)
  var = jnp.mean((z - mean) ** 2, axis=0, keepdims=True)
  h = (z - mean) * lax.rsqrt(var + EPS_BN) * g_ref[...] + be_ref[...]
  h = jnp.maximum(h, 0.0)
  o_ref[...] = lax.dot_general(
      h, w_ref[...], (((1,), (1,)), ((), ())),
      preferred_element_type=jnp.float32, precision=lax.Precision.HIGHEST)


def _tc_layer(y, acc, b, g, be, w_next):
  return pl.pallas_call(
      _layer_body,
      out_shape=jax.ShapeDtypeStruct((N, D), jnp.float32),
  )(y, acc, b, g, be, w_next)


def _final_body(y_ref, acc_ref, b_ref, o_ref):
  o_ref[...] = y_ref[...] + acc_ref[0] + acc_ref[1] + b_ref[...]


def _tc_final(y, acc, b):
  return pl.pallas_call(
      _final_body,
      out_shape=jax.ShapeDtypeStruct((N, D), jnp.float32),
  )(y, acc, b)


def kernel(x, edge_index, Ws, bs, gammas, betas):
  src_r = edge_index[0].reshape(NW, NCHUNK, CH)
  dst_r = edge_index[1].reshape(NW, NCHUNK, CH)
  zeros = jnp.zeros((N, D), jnp.float32)

  pad = D - NUM_CLASSES
  w4 = jnp.concatenate([Ws[4], jnp.zeros((pad, D), jnp.float32)], axis=0)
  b4 = jnp.concatenate([bs[4], jnp.zeros((pad,), jnp.float32)]).reshape(1, D)
  w_next = [Ws[1], Ws[2], Ws[3], w4]

  y = _input_matmul(x, Ws[0])
  for i in range(4):
    acc = _sc_aggregate_call(y, src_r, dst_r, zeros)
    y = _tc_layer(y, acc, bs[i].reshape(1, D), gammas[i].reshape(1, D),
                  betas[i].reshape(1, D), w_next[i])
  acc = _sc_aggregate_call(y, src_r, dst_r, zeros)
  z = _tc_final(y, acc, b4)
  return z[:, :NUM_CLASSES]


# speedup vs baseline: 10.0197x; 1.1966x over previous
import functools

import jax
import jax.numpy as jnp
from jax import lax
from jax.experimental import pallas as pl
from jax.experimental.pallas import tpu as pltpu
from jax.experimental.pallas import tpu_sc as plsc

N = 10000
E = 320000
D = 64
NUM_CLASSES = 47
EPS_BN = 1e-5

NC, NS = 2, 16
NW = NC * NS
EPW = E // NW
CH = 80
NCHUNK = EPW // CH
RPT = 624
TAIL = N - NS * RPT


def _sc_aggregate_call(y, src_r, dst_r, zeros):
  mesh = plsc.VectorSubcoreMesh(core_axis_name="c", subcore_axis_name="s")

  @functools.partial(
      pl.kernel,
      out_type=jax.ShapeDtypeStruct((NC, N, D), jnp.float32),
      mesh=mesh,
      scratch_types=[
          pltpu.VMEM((NCHUNK, CH), jnp.int32),
          pltpu.VMEM((NCHUNK, CH), jnp.int32),
          pltpu.VMEM((2, CH, D), jnp.float32),
          pltpu.VMEM_SHARED((N, D), jnp.float32),
          pltpu.SemaphoreType.DMA((2,)),
      ],
      compiler_params=pltpu.CompilerParams(use_tc_tiling_on_sc=False),
  )
  def agg(y_hbm, src_hbm, dst_hbm, zeros_hbm, out_hbm, src_v, dst_v, rows_v,
          acc_s, sem):
    cid = lax.axis_index("c")
    sid = lax.axis_index("s")
    wid = sid * NC + cid

    pltpu.sync_copy(zeros_hbm.at[pl.ds(sid * RPT, RPT)],
                    acc_s.at[pl.ds(sid * RPT, RPT)])

    @pl.when(sid == NS - 1)
    def _seed_tail():
      pltpu.sync_copy(zeros_hbm.at[pl.ds(NS * RPT, TAIL)],
                      acc_s.at[pl.ds(NS * RPT, TAIL)])

    pltpu.sync_copy(src_hbm.at[wid], src_v)
    pltpu.sync_copy(dst_hbm.at[wid], dst_v)
    plsc.subcore_barrier()

    pltpu.async_copy(y_hbm.at[src_v.at[0]], rows_v.at[0], sem.at[0])

    def step(j, carry):
      b = lax.rem(j, 2)
      pltpu.make_async_copy(y_hbm.at[src_v.at[j]], rows_v.at[b],
                            sem.at[b]).wait()

      @pl.when(j < NCHUNK - 1)
      def _prefetch():
        nb = 1 - b
        pltpu.async_copy(y_hbm.at[src_v.at[j + 1]], rows_v.at[nb], sem.at[nb])

      pltpu.sync_copy(rows_v.at[b], acc_s.at[dst_v.at[j]], add=True)
      return carry

    lax.fori_loop(0, NCHUNK, step, 0, unroll=False)

    plsc.subcore_barrier()
    pltpu.sync_copy(acc_s.at[pl.ds(sid * RPT, RPT)],
                    out_hbm.at[cid, pl.ds(sid * RPT, RPT)])

    @pl.when(sid == NS - 1)
    def _write_tail():
      pltpu.sync_copy(acc_s.at[pl.ds(NS * RPT, TAIL)],
                      out_hbm.at[cid, pl.ds(NS * RPT, TAIL)])

  return agg(y, src_r, dst_r, zeros)


def _mm_body(x_ref, w_ref, o_ref):
  o_ref[...] = lax.dot_general(
      x_ref[...], w_ref[...], (((1,), (1,)), ((), ())),
      preferred_element_type=jnp.float32, precision=lax.Precision.HIGHEST)


def _input_matmul(x, w):
  return pl.pallas_call(
      _mm_body,
      out_shape=jax.ShapeDtypeStruct((N, D), jnp.float32),
  )(x, w)


def _layer_body(y_ref, acc_ref, b_ref, g_ref, be_ref, w_ref, o_ref):
  z = y_ref[...] + acc_ref[0] + acc_ref[1] + b_ref[...]
  mean = jnp.mean(z, axis=0, keepdims=True)
  var = jnp.mean((z - mean) ** 2, axis=0, keepdims=True)
  h = (z - mean) * lax.rsqrt(var + EPS_BN) * g_ref[...] + be_ref[...]
  h = jnp.maximum(h, 0.0)
  o_ref[...] = lax.dot_general(
      h, w_ref[...], (((1,), (1,)), ((), ())),
      preferred_element_type=jnp.float32, precision=lax.Precision.HIGHEST)


def _tc_layer(y, acc, b, g, be, w_next):
  return pl.pallas_call(
      _layer_body,
      out_shape=jax.ShapeDtypeStruct((N, D), jnp.float32),
  )(y, acc, b, g, be, w_next)


def _final_body(y_ref, acc_ref, b_ref, o_ref):
  o_ref[...] = y_ref[...] + acc_ref[0] + acc_ref[1] + b_ref[...]


def _tc_final(y, acc, b):
  return pl.pallas_call(
      _final_body,
      out_shape=jax.ShapeDtypeStruct((N, D), jnp.float32),
  )(y, acc, b)


def kernel(x, edge_index, Ws, bs, gammas, betas):
  src_r = edge_index[0].reshape(NW, NCHUNK, CH)
  dst_r = edge_index[1].reshape(NW, NCHUNK, CH)
  zeros = jnp.zeros((N, D), jnp.float32)

  pad = D - NUM_CLASSES
  w4 = jnp.concatenate([Ws[4], jnp.zeros((pad, D), jnp.float32)], axis=0)
  b4 = jnp.concatenate([bs[4], jnp.zeros((pad,), jnp.float32)]).reshape(1, D)
  w_next = [Ws[1], Ws[2], Ws[3], w4]

  y = _input_matmul(x, Ws[0])
  for i in range(4):
    acc = _sc_aggregate_call(y, src_r, dst_r, zeros)
    y = _tc_layer(y, acc, bs[i].reshape(1, D), gammas[i].reshape(1, D),
                  betas[i].reshape(1, D), w_next[i])
  acc = _sc_aggregate_call(y, src_r, dst_r, zeros)
  z = _tc_final(y, acc, b4)
  return z[:, :NUM_CLASSES]


# speedup vs baseline: 12.9828x; 1.2957x over previous
import functools

import jax
import jax.numpy as jnp
from jax import lax
from jax.experimental import pallas as pl
from jax.experimental.pallas import tpu as pltpu
from jax.experimental.pallas import tpu_sc as plsc

N = 10000
E = 320000
D = 64
NUM_CLASSES = 47
EPS_BN = 1e-5

NC, NS = 2, 16
NW = NC * NS
EPW = E // NW
CH = 80
NCHUNK = EPW // CH
RPT = 624
TAIL = N - NS * RPT


def _sc_aggregate_call(y, src_r, dst_r, zeros):
  mesh = plsc.VectorSubcoreMesh(core_axis_name="c", subcore_axis_name="s")

  @functools.partial(
      pl.kernel,
      out_type=jax.ShapeDtypeStruct((NC, N, D), jnp.float32),
      mesh=mesh,
      scratch_types=[
          pltpu.VMEM((NCHUNK, CH), jnp.int32),
          pltpu.VMEM((NCHUNK, CH), jnp.int32),
          pltpu.VMEM((2, CH, D), jnp.float32),
          pltpu.VMEM_SHARED((N, D), jnp.float32),
          pltpu.VMEM_SHARED((N, D), jnp.float32),
          pltpu.SemaphoreType.DMA((2,)),
      ],
      compiler_params=pltpu.CompilerParams(use_tc_tiling_on_sc=False),
  )
  def agg(y_hbm, src_hbm, dst_hbm, zeros_hbm, out_hbm, src_v, dst_v, rows_v,
          acc_s, y_s, sem):
    cid = lax.axis_index("c")
    sid = lax.axis_index("s")
    wid = sid * NC + cid

    pltpu.sync_copy(zeros_hbm.at[pl.ds(sid * RPT, RPT)],
                    acc_s.at[pl.ds(sid * RPT, RPT)])
    pltpu.sync_copy(y_hbm.at[pl.ds(sid * RPT, RPT)],
                    y_s.at[pl.ds(sid * RPT, RPT)])

    @pl.when(sid == NS - 1)
    def _seed_tail():
      pltpu.sync_copy(zeros_hbm.at[pl.ds(NS * RPT, TAIL)],
                      acc_s.at[pl.ds(NS * RPT, TAIL)])
      pltpu.sync_copy(y_hbm.at[pl.ds(NS * RPT, TAIL)],
                      y_s.at[pl.ds(NS * RPT, TAIL)])

    pltpu.sync_copy(src_hbm.at[wid], src_v)
    pltpu.sync_copy(dst_hbm.at[wid], dst_v)
    plsc.subcore_barrier()

    pltpu.async_copy(y_s.at[src_v.at[0]], rows_v.at[0], sem.at[0])

    def step(j, carry):
      b = lax.rem(j, 2)
      pltpu.make_async_copy(y_s.at[src_v.at[j]], rows_v.at[b],
                            sem.at[b]).wait()

      @pl.when(j < NCHUNK - 1)
      def _prefetch():
        nb = 1 - b
        pltpu.async_copy(y_s.at[src_v.at[j + 1]], rows_v.at[nb], sem.at[nb])

      pltpu.sync_copy(rows_v.at[b], acc_s.at[dst_v.at[j]], add=True)
      return carry

    lax.fori_loop(0, NCHUNK, step, 0, unroll=False)

    plsc.subcore_barrier()
    pltpu.sync_copy(acc_s.at[pl.ds(sid * RPT, RPT)],
                    out_hbm.at[cid, pl.ds(sid * RPT, RPT)])

    @pl.when(sid == NS - 1)
    def _write_tail():
      pltpu.sync_copy(acc_s.at[pl.ds(NS * RPT, TAIL)],
                      out_hbm.at[cid, pl.ds(NS * RPT, TAIL)])

  return agg(y, src_r, dst_r, zeros)


def _mm_body(x_ref, w_ref, o_ref):
  o_ref[...] = lax.dot_general(
      x_ref[...], w_ref[...], (((1,), (1,)), ((), ())),
      preferred_element_type=jnp.float32, precision=lax.Precision.HIGHEST)


def _input_matmul(x, w):
  return pl.pallas_call(
      _mm_body,
      out_shape=jax.ShapeDtypeStruct((N, D), jnp.float32),
  )(x, w)


def _layer_body(y_ref, acc_ref, b_ref, g_ref, be_ref, w_ref, o_ref):
  z = y_ref[...] + acc_ref[0] + acc_ref[1] + b_ref[...]
  mean = jnp.mean(z, axis=0, keepdims=True)
  var = jnp.mean((z - mean) ** 2, axis=0, keepdims=True)
  h = (z - mean) * lax.rsqrt(var + EPS_BN) * g_ref[...] + be_ref[...]
  h = jnp.maximum(h, 0.0)
  o_ref[...] = lax.dot_general(
      h, w_ref[...], (((1,), (1,)), ((), ())),
      preferred_element_type=jnp.float32, precision=lax.Precision.HIGHEST)


def _tc_layer(y, acc, b, g, be, w_next):
  return pl.pallas_call(
      _layer_body,
      out_shape=jax.ShapeDtypeStruct((N, D), jnp.float32),
  )(y, acc, b, g, be, w_next)


def _final_body(y_ref, acc_ref, b_ref, o_ref):
  o_ref[...] = y_ref[...] + acc_ref[0] + acc_ref[1] + b_ref[...]


def _tc_final(y, acc, b):
  return pl.pallas_call(
      _final_body,
      out_shape=jax.ShapeDtypeStruct((N, D), jnp.float32),
  )(y, acc, b)


def kernel(x, edge_index, Ws, bs, gammas, betas):
  src_r = edge_index[0].reshape(NW, NCHUNK, CH)
  dst_r = edge_index[1].reshape(NW, NCHUNK, CH)
  zeros = jnp.zeros((N, D), jnp.float32)

  pad = D - NUM_CLASSES
  w4 = jnp.concatenate([Ws[4], jnp.zeros((pad, D), jnp.float32)], axis=0)
  b4 = jnp.concatenate([bs[4], jnp.zeros((pad,), jnp.float32)]).reshape(1, D)
  w_next = [Ws[1], Ws[2], Ws[3], w4]

  y = _input_matmul(x, Ws[0])
  for i in range(4):
    acc = _sc_aggregate_call(y, src_r, dst_r, zeros)
    y = _tc_layer(y, acc, bs[i].reshape(1, D), gammas[i].reshape(1, D),
                  betas[i].reshape(1, D), w_next[i])
  acc = _sc_aggregate_call(y, src_r, dst_r, zeros)
  z = _tc_final(y, acc, b4)
  return z[:, :NUM_CLASSES]
